# Initial kernel scaffold; baseline (speedup 1.0000x reference)
import jax
import jax.numpy as jnp
from jax.experimental import pallas as pl


def kernel(x, weight, bias):
    raise NotImplementedError("write your pallas kernel here")


# speedup vs baseline: 1.4522x; 1.4522x over previous
import functools

import jax
import jax.numpy as jnp
from jax.experimental import pallas as pl

TOPK = 8
E = 64
BM = 512


def _router_kernel(x_ref, wt_ref, b_ref, w_out_ref, i_out_ref):
    logits = jnp.dot(x_ref[...], wt_ref[...], preferred_element_type=jnp.float32)
    scores = jax.nn.sigmoid(logits)
    biased = scores + b_ref[...]
    cols = jax.lax.broadcasted_iota(jnp.int32, biased.shape, 1)

    idx_parts = []
    w_parts = []
    cur = biased
    for _ in range(TOPK):
        m = jnp.max(cur, axis=1, keepdims=True)
        is_max = cur == m
        idx_k = jnp.min(jnp.where(is_max, cols, E), axis=1, keepdims=True)
        sel = cols == idx_k
        w_k = jnp.sum(jnp.where(sel, scores, 0.0), axis=1, keepdims=True)
        idx_parts.append(idx_k)
        w_parts.append(w_k)
        cur = jnp.where(sel, -jnp.inf, cur)

    w = jnp.concatenate(w_parts, axis=1)
    idx = jnp.concatenate(idx_parts, axis=1)
    w = w / (jnp.sum(w, axis=1, keepdims=True) + 1e-20)
    w_out_ref[...] = w
    i_out_ref[...] = idx


@functools.partial(jax.jit, static_argnames=())
def kernel(x, weight, bias):
    t = x.shape[0]
    wt = weight.T
    b2 = bias.reshape(1, E)
    grid = (t // BM,)
    w, idx = pl.pallas_call(
        _router_kernel,
        grid=grid,
        in_specs=[
            pl.BlockSpec((BM, x.shape[1]), lambda i: (i, 0)),
            pl.BlockSpec((x.shape[1], E), lambda i: (0, 0)),
            pl.BlockSpec((1, E), lambda i: (0, 0)),
        ],
        out_specs=[
            pl.BlockSpec((BM, TOPK), lambda i: (i, 0)),
            pl.BlockSpec((BM, TOPK), lambda i: (i, 0)),
        ],
        out_shape=[
            jax.ShapeDtypeStruct((t, TOPK), jnp.float32),
            jax.ShapeDtypeStruct((t, TOPK), jnp.int32),
        ],
    )(x, wt, b2)
    return w, idx


# speedup vs baseline: 2.1489x; 1.4797x over previous
import functools

import jax
import jax.numpy as jnp
from jax.experimental import pallas as pl

TOPK = 8
E = 64
BM = 512
NEG = -3.0e38


def _router_kernel(x_ref, wt_ref, b_ref, w_out_ref, i_out_ref):
    logits = jnp.dot(x_ref[...], wt_ref[...], preferred_element_type=jnp.float32)
    lt = logits.T
    scores = jax.nn.sigmoid(lt)
    biased = scores + b_ref[...]
    rows = jax.lax.broadcasted_iota(jnp.int32, biased.shape, 0).astype(jnp.float32)

    idx_parts = []
    w_parts = []
    cur = biased
    for _ in range(TOPK):
        m = jnp.max(cur, axis=0, keepdims=True)
        is_max = cur == m
        idx_k = jnp.min(jnp.where(is_max, rows, float(E)), axis=0, keepdims=True)
        sel = rows == idx_k
        w_k = jnp.sum(jnp.where(sel, scores, 0.0), axis=0, keepdims=True)
        idx_parts.append(idx_k)
        w_parts.append(w_k)
        cur = jnp.where(sel, NEG, cur)

    w = jnp.concatenate(w_parts, axis=0)
    idx = jnp.concatenate(idx_parts, axis=0)
    w = w / (jnp.sum(w, axis=0, keepdims=True) + 1e-20)
    w_out_ref[...] = w
    i_out_ref[...] = idx.astype(jnp.int32)


@functools.partial(jax.jit, static_argnames=())
def kernel(x, weight, bias):
    t = x.shape[0]
    wt = weight.T
    bt = bias.reshape(E, 1)
    grid = (t // BM,)
    w_t, idx_t = pl.pallas_call(
        _router_kernel,
        grid=grid,
        in_specs=[
            pl.BlockSpec((BM, x.shape[1]), lambda i: (i, 0)),
            pl.BlockSpec((x.shape[1], E), lambda i: (0, 0)),
            pl.BlockSpec((E, 1), lambda i: (0, 0)),
        ],
        out_specs=[
            pl.BlockSpec((TOPK, BM), lambda i: (0, i)),
            pl.BlockSpec((TOPK, BM), lambda i: (0, i)),
        ],
        out_shape=[
            jax.ShapeDtypeStruct((TOPK, t), jnp.float32),
            jax.ShapeDtypeStruct((TOPK, t), jnp.int32),
        ],
    )(x, wt, bt)
    return w_t.T, idx_t.T


# speedup vs baseline: 2.2634x; 1.0533x over previous
import functools

import jax
import jax.numpy as jnp
from jax.experimental import pallas as pl

TOPK = 8
E = 64
BM = 1024
NEG = -3.0e38


def _router_kernel(x_ref, wt_ref, b_ref, w_out_ref, i_out_ref):
    logits = jnp.dot(x_ref[...], wt_ref[...], preferred_element_type=jnp.float32)
    lt = logits.T
    scores = jax.nn.sigmoid(lt)
    biased = scores + b_ref[...]
    rows = jax.lax.broadcasted_iota(jnp.int32, biased.shape, 0).astype(jnp.float32)

    idx_parts = []
    w_parts = []
    cur = biased
    for _ in range(TOPK):
        m = jnp.max(cur, axis=0, keepdims=True)
        is_max = cur == m
        idx_k = jnp.min(jnp.where(is_max, rows, float(E)), axis=0, keepdims=True)
        sel = rows == idx_k
        w_k = jnp.sum(jnp.where(sel, scores, 0.0), axis=0, keepdims=True)
        idx_parts.append(idx_k)
        w_parts.append(w_k)
        cur = jnp.where(sel, NEG, cur)

    w = jnp.concatenate(w_parts, axis=0)
    idx = jnp.concatenate(idx_parts, axis=0)
    w = w / (jnp.sum(w, axis=0, keepdims=True) + 1e-20)
    w_out_ref[...] = w
    i_out_ref[...] = idx.astype(jnp.int32)


@functools.partial(jax.jit, static_argnames=())
def kernel(x, weight, bias):
    t = x.shape[0]
    wt = weight.T
    bt = bias.reshape(E, 1)
    grid = (t // BM,)
    w_t, idx_t = pl.pallas_call(
        _router_kernel,
        grid=grid,
        in_specs=[
            pl.BlockSpec((BM, x.shape[1]), lambda i: (i, 0)),
            pl.BlockSpec((x.shape[1], E), lambda i: (0, 0)),
            pl.BlockSpec((E, 1), lambda i: (0, 0)),
        ],
        out_specs=[
            pl.BlockSpec((TOPK, BM), lambda i: (0, i)),
            pl.BlockSpec((TOPK, BM), lambda i: (0, i)),
        ],
        out_shape=[
            jax.ShapeDtypeStruct((TOPK, t), jnp.float32),
            jax.ShapeDtypeStruct((TOPK, t), jnp.int32),
        ],
    )(x, wt, bt)
    return w_t.T, idx_t.T
